# Initial kernel scaffold; baseline (speedup 1.0000x reference)
import functools

import jax
import jax.numpy as jnp
from jax import lax
from jax.experimental import pallas as pl
from jax.experimental.pallas import tpu as pltpu
from jax.experimental.pallas import tpu_sc as plsc

VOCAB = 100000
SUB = 128
NUM_TABLES = 8

_info = plsc.get_sparse_core_info()
_NC, _NS, _L = _info.num_cores, _info.num_subcores, _info.num_lanes
_NW = _NC * _NS

_ROWS_TOTAL = 4 * 2048 * 8
_ROWS_PER_W = _ROWS_TOTAL // _NW
_CHUNK = 128
_NCHUNK = _ROWS_PER_W // _CHUNK


def _sc_gather(out_rows):
    mesh = plsc.VectorSubcoreMesh(core_axis_name="c", subcore_axis_name="s")

    @functools.partial(
        pl.kernel,
        mesh=mesh,
        out_type=out_rows,
        scratch_types=[
            pltpu.VMEM((_NCHUNK, _CHUNK), jnp.int32),
            pltpu.VMEM((_CHUNK, SUB), jnp.float32),
            pltpu.VMEM((_CHUNK, SUB), jnp.float32),
            pltpu.SemaphoreType.DMA,
            pltpu.SemaphoreType.DMA,
        ],
    )
    def k(x_hbm, tbl_hbm, out_hbm, idx_v, rows_a, rows_b, sem_a, sem_b):
        wid = lax.axis_index("s") * _NC + lax.axis_index("c")
        base = wid * _ROWS_PER_W

        pltpu.sync_copy(x_hbm.at[wid], idx_v)

        offv = (lax.iota(jnp.int32, (_L,)) % NUM_TABLES) * VOCAB
        for c in range(_NCHUNK):
            for g in range(_CHUNK // _L):
                sl = pl.ds(g * _L, _L)
                idx_v[c, sl] = idx_v[c, sl] + offv

        bufs = (rows_a, rows_b)
        sems = (sem_a, sem_b)
        copies = [pltpu.async_copy(tbl_hbm.at[idx_v.at[0]], bufs[0], sems[0])]
        for c in range(_NCHUNK):
            nxt = c + 1
            if nxt < _NCHUNK:
                copies.append(
                    pltpu.async_copy(
                        tbl_hbm.at[idx_v.at[nxt]], bufs[nxt % 2], sems[nxt % 2]
                    )
                )
            copies[c].wait()
            pltpu.sync_copy(bufs[c % 2], out_hbm.at[pl.ds(base + c * _CHUNK, _CHUNK)])

    return k


def kernel(x, tables):
    B, S, T = x.shape
    x_flat3 = x.reshape(_NW, _NCHUNK, _CHUNK).astype(jnp.int32)
    big_table = tables.reshape(NUM_TABLES * VOCAB, SUB)
    out_rows = jax.ShapeDtypeStruct((_ROWS_TOTAL, SUB), jnp.float32)
    out = _sc_gather(out_rows)(x_flat3, big_table)
    return out.reshape(B, S, T * SUB)


# speedup vs baseline: 4.3152x; 4.3152x over previous
import functools

import jax
import jax.numpy as jnp
from jax import lax
from jax.experimental import pallas as pl
from jax.experimental.pallas import tpu as pltpu
from jax.experimental.pallas import tpu_sc as plsc

VOCAB = 100000
SUB = 128
NUM_TABLES = 8

_info = plsc.get_sparse_core_info()
_NC, _NS, _L = _info.num_cores, _info.num_subcores, _info.num_lanes
_NW = _NC * _NS

_ROWS_TOTAL = 4 * 2048 * 8
_ROWS_PER_W = _ROWS_TOTAL // _NW
_CHUNK = 128
_NCHUNK = _ROWS_PER_W // _CHUNK


def _sc_gather(out_rows):
    mesh = plsc.VectorSubcoreMesh(core_axis_name="c", subcore_axis_name="s")

    @functools.partial(
        pl.kernel,
        mesh=mesh,
        out_type=out_rows,
        scratch_types=[
            pltpu.VMEM((_NCHUNK, _CHUNK), jnp.int32),
            pltpu.VMEM((_CHUNK, SUB), jnp.float32),
            pltpu.VMEM((_CHUNK, SUB), jnp.float32),
            pltpu.SemaphoreType.DMA,
            pltpu.SemaphoreType.DMA,
        ],
    )
    def k(x_hbm, tbl_hbm, out_hbm, idx_v, rows_a, rows_b, sem_a, sem_b):
        wid = lax.axis_index("s") * _NC + lax.axis_index("c")
        base = wid * _ROWS_PER_W

        pltpu.sync_copy(x_hbm.at[wid], idx_v)

        offv = (lax.iota(jnp.int32, _L) % NUM_TABLES) * VOCAB
        for c in range(_NCHUNK):
            for g in range(_CHUNK // _L):
                sl = pl.ds(g * _L, _L)
                idx_v[c, sl] = idx_v[c, sl] + offv

        bufs = (rows_a, rows_b)
        sems = (sem_a, sem_b)
        copies = [pltpu.async_copy(tbl_hbm.at[idx_v.at[0]], bufs[0], sems[0])]
        for c in range(_NCHUNK):
            nxt = c + 1
            if nxt < _NCHUNK:
                copies.append(
                    pltpu.async_copy(
                        tbl_hbm.at[idx_v.at[nxt]], bufs[nxt % 2], sems[nxt % 2]
                    )
                )
            copies[c].wait()
            pltpu.sync_copy(bufs[c % 2], out_hbm.at[pl.ds(base + c * _CHUNK, _CHUNK)])

    return k


def kernel(x, tables):
    B, S, T = x.shape
    x_flat3 = x.reshape(_NW, _NCHUNK, _CHUNK).astype(jnp.int32)
    big_table = tables.reshape(NUM_TABLES * VOCAB, SUB)
    out_rows = jax.ShapeDtypeStruct((_ROWS_TOTAL, SUB), jnp.float32)
    out = _sc_gather(out_rows)(x_flat3, big_table)
    return out.reshape(B, S, T * SUB)


# speedup vs baseline: 4.4096x; 1.0219x over previous
import functools

import jax
import jax.numpy as jnp
from jax import lax
from jax.experimental import pallas as pl
from jax.experimental.pallas import tpu as pltpu
from jax.experimental.pallas import tpu_sc as plsc

VOCAB = 100000
SUB = 128
NUM_TABLES = 8

_info = plsc.get_sparse_core_info()
_NC, _NS, _L = _info.num_cores, _info.num_subcores, _info.num_lanes
_NW = _NC * _NS

_ROWS_TOTAL = 4 * 2048 * 8
_ROWS_PER_W = _ROWS_TOTAL // _NW
_CHUNK = 128
_NCHUNK = _ROWS_PER_W // _CHUNK
_NBUF = 6


def _sc_gather(out_rows):
    mesh = plsc.VectorSubcoreMesh(core_axis_name="c", subcore_axis_name="s")

    @functools.partial(
        pl.kernel,
        mesh=mesh,
        out_type=out_rows,
        scratch_types=(
            [pltpu.VMEM((_NCHUNK, _CHUNK), jnp.int32)]
            + [pltpu.VMEM((_CHUNK, SUB), jnp.float32) for _ in range(_NBUF)]
            + [pltpu.SemaphoreType.DMA for _ in range(2 * _NBUF)]
        ),
    )
    def k(x_hbm, tbl_hbm, out_hbm, idx_v, *rest):
        bufs = rest[:_NBUF]
        sem_g = rest[_NBUF:2 * _NBUF]
        sem_w = rest[2 * _NBUF:]
        wid = lax.axis_index("s") * _NC + lax.axis_index("c")
        base = wid * _ROWS_PER_W

        pltpu.sync_copy(x_hbm.at[wid], idx_v)

        offv = (lax.iota(jnp.int32, _L) % NUM_TABLES) * VOCAB

        def add_off(c):
            for g in range(_CHUNK // _L):
                sl = pl.ds(g * _L, _L)
                idx_v[c, sl] = idx_v[c, sl] + offv

        def start_gather(c):
            return pltpu.async_copy(
                tbl_hbm.at[idx_v.at[c]], bufs[c % _NBUF], sem_g[c % _NBUF]
            )

        gcp = [None] * _NCHUNK
        wcp = [None] * _NCHUNK
        for c in range(_NBUF):
            add_off(c)
            gcp[c] = start_gather(c)
        for c in range(_NCHUNK):
            gcp[c].wait()
            wcp[c] = pltpu.async_copy(
                bufs[c % _NBUF],
                out_hbm.at[pl.ds(base + c * _CHUNK, _CHUNK)],
                sem_w[c % _NBUF],
            )
            n = c + _NBUF
            if n < _NCHUNK:
                add_off(n)
                wcp[c].wait()
                gcp[n] = start_gather(n)
        for c in range(_NCHUNK - _NBUF, _NCHUNK):
            wcp[c].wait()

    return k


def kernel(x, tables):
    B, S, T = x.shape
    x_flat3 = x.reshape(_NW, _NCHUNK, _CHUNK).astype(jnp.int32)
    big_table = tables.reshape(NUM_TABLES * VOCAB, SUB)
    out_rows = jax.ShapeDtypeStruct((_ROWS_TOTAL, SUB), jnp.float32)
    out = _sc_gather(out_rows)(x_flat3, big_table)
    return out.reshape(B, S, T * SUB)


# speedup vs baseline: 6.7373x; 1.5279x over previous
import functools

import jax
import jax.numpy as jnp
from jax import lax
from jax.experimental import pallas as pl
from jax.experimental.pallas import tpu as pltpu
from jax.experimental.pallas import tpu_sc as plsc

VOCAB = 100000
SUB = 128
NUM_TABLES = 8

_info = plsc.get_sparse_core_info()
_NC, _NS, _L = _info.num_cores, _info.num_subcores, _info.num_lanes
_NW = _NC * _NS

_ROWS_TOTAL = 4 * 2048 * 8
_ROWS_PER_W = _ROWS_TOTAL // _NW
_CHUNK = 128
_NCHUNK = _ROWS_PER_W // _CHUNK
_NBUF = 6
_NGRP = _CHUNK // _L


def _sc_gather(out_rows):
    mesh = plsc.VectorSubcoreMesh(core_axis_name="c", subcore_axis_name="s")

    @functools.partial(
        pl.kernel,
        mesh=mesh,
        out_type=out_rows,
        scratch_types=(
            [
                pltpu.VMEM((_NCHUNK, _CHUNK), jnp.int32),
                pltpu.VMEM((_NCHUNK, _CHUNK), jnp.int32),
            ]
            + [pltpu.VMEM((_CHUNK, SUB), jnp.float32) for _ in range(_NBUF)]
            + [pltpu.SemaphoreType.DMA for _ in range(2 * _NBUF)]
        ),
    )
    def k(x_hbm, tbl_hbm, out_hbm, xin_v, idx_v, *rest):
        bufs = rest[:_NBUF]
        sem_g = rest[_NBUF:2 * _NBUF]
        sem_w = rest[2 * _NBUF:]
        wid = lax.axis_index("s") * _NC + lax.axis_index("c")
        base = wid * _ROWS_PER_W

        pltpu.sync_copy(x_hbm.at[wid], xin_v)

        io = lax.iota(jnp.int32, _L)
        soff_m = [
            jnp.where(io < 8, 2 * m * VOCAB, (2 * m + 1) * VOCAB)
            for m in range(4)
        ]

        def fill(c):
            for g in range(_NGRP):
                sl = pl.ds(g * _L, _L)
                idx_v[c, sl] = xin_v[c, sl] + soff_m[(c * _NGRP + g) % 4]

        def start_gather(c):
            return pltpu.async_copy(
                tbl_hbm.at[idx_v.at[c]], bufs[c % _NBUF], sem_g[c % _NBUF]
            )

        gcp = [None] * _NCHUNK
        wcp = [None] * _NCHUNK
        for c in range(_NBUF):
            fill(c)
            gcp[c] = start_gather(c)
        for c in range(_NCHUNK):
            gcp[c].wait()
            wcp[c] = pltpu.async_copy(
                bufs[c % _NBUF],
                out_hbm.at[pl.ds(base + c * _CHUNK, _CHUNK)],
                sem_w[c % _NBUF],
            )
            n = c + _NBUF
            if n < _NCHUNK:
                fill(n)
                wcp[c].wait()
                gcp[n] = start_gather(n)
        for c in range(_NCHUNK - _NBUF, _NCHUNK):
            wcp[c].wait()

    return k


def kernel(x, tables):
    B, S, T = x.shape
    xq = (
        x.astype(jnp.int32)
        .reshape(B * S // 8, 8, T)
        .transpose(0, 2, 1)
        .reshape(_NW, _NCHUNK, _CHUNK)
    )
    big_table = tables.reshape(NUM_TABLES * VOCAB, SUB)
    out_rows = jax.ShapeDtypeStruct((_ROWS_TOTAL, SUB), jnp.float32)
    out = _sc_gather(out_rows)(xq, big_table)
    return (
        out.reshape(B * S // 8, 8, 8, SUB)
        .transpose(0, 2, 1, 3)
        .reshape(B, S, T * SUB)
    )


# speedup vs baseline: 7.2036x; 1.0692x over previous
import functools

import jax
import jax.numpy as jnp
from jax import lax
from jax.experimental import pallas as pl
from jax.experimental.pallas import tpu as pltpu
from jax.experimental.pallas import tpu_sc as plsc

VOCAB = 100000
SUB = 128
NUM_TABLES = 8

_info = plsc.get_sparse_core_info()
_NC, _NS, _L = _info.num_cores, _info.num_subcores, _info.num_lanes
_NW = _NC * _NS

_ROWS_TOTAL = 4 * 2048 * 8
_ROWS_PER_W = _ROWS_TOTAL // _NW
_CHUNK = 128
_NCHUNK = _ROWS_PER_W // _CHUNK
_NBUF = 6
_NGRP = _CHUNK // _L


def _sc_gather(out_rows):
    mesh = plsc.VectorSubcoreMesh(core_axis_name="c", subcore_axis_name="s")

    @functools.partial(
        pl.kernel,
        mesh=mesh,
        out_type=out_rows,
        compiler_params=pltpu.CompilerParams(needs_layout_passes=False),
        scratch_types=(
            [
                pltpu.VMEM((2, NUM_TABLES, SUB), jnp.int32),
                pltpu.VMEM((_NCHUNK, _CHUNK), jnp.int32),
            ]
            + [pltpu.VMEM((_CHUNK, SUB), jnp.float32) for _ in range(_NBUF)]
            + [pltpu.SemaphoreType.DMA for _ in range(2 * _NBUF)]
        ),
    )
    def k(x_hbm, tbl_hbm, out_hbm, xin_v, idx_v, *rest):
        bufs = rest[:_NBUF]
        sem_g = rest[_NBUF:2 * _NBUF]
        sem_w = rest[2 * _NBUF:]
        wid = lax.axis_index("s") * _NC + lax.axis_index("c")
        base = wid * _ROWS_PER_W

        pltpu.sync_copy(x_hbm.at[wid // 8, pl.ds((wid % 8) * 2, 2)], xin_v)

        io = lax.iota(jnp.int32, _L)
        lo = io % 8
        hi = jnp.where(io < 8, 0, 1)
        s_m = [hi + 2 * m for m in range(4)]
        soff_m = [
            jnp.where(io < 8, 2 * m * VOCAB, (2 * m + 1) * VOCAB)
            for m in range(4)
        ]

        def fill(c):
            for g in range(_NGRP):
                G = c * _NGRP + g
                m = G % 4
                a = (G >> 2) >> 4
                cc = ((G >> 2) & 15) * 8
                va = jnp.full((_L,), a, jnp.int32)
                gg = plsc.load_gather(xin_v, [va, s_m[m], lo + cc])
                idx_v[c, pl.ds(g * _L, _L)] = gg + soff_m[m]

        def start_gather(c):
            return pltpu.async_copy(
                tbl_hbm.at[idx_v.at[c]], bufs[c % _NBUF], sem_g[c % _NBUF]
            )

        gcp = [None] * _NCHUNK
        wcp = [None] * _NCHUNK
        for c in range(_NBUF):
            fill(c)
            gcp[c] = start_gather(c)
        for c in range(_NCHUNK):
            gcp[c].wait()
            wcp[c] = pltpu.async_copy(
                bufs[c % _NBUF],
                out_hbm.at[pl.ds(base + c * _CHUNK, _CHUNK)],
                sem_w[c % _NBUF],
            )
            n = c + _NBUF
            if n < _NCHUNK:
                fill(n)
                wcp[c].wait()
                gcp[n] = start_gather(n)
        for c in range(_NCHUNK - _NBUF, _NCHUNK):
            wcp[c].wait()

    return k


def kernel(x, tables):
    B, S, T = x.shape
    xv = (
        x.astype(jnp.int32)
        .transpose(0, 2, 1)
        .reshape(B, T, S // SUB, SUB)
        .transpose(0, 2, 1, 3)
    )
    big_table = tables.reshape(NUM_TABLES * VOCAB, SUB)
    out_rows = jax.ShapeDtypeStruct((_ROWS_TOTAL, SUB), jnp.float32)
    out = _sc_gather(out_rows)(xv, big_table)
    return (
        out.reshape(B * S // 8, 8, 8, SUB)
        .transpose(0, 2, 1, 3)
        .reshape(B, S, T * SUB)
    )
